# baseline (speedup 1.0000x reference)
import functools

import numpy as np
import jax
import jax.numpy as jnp
from jax import lax
from jax.experimental import pallas as pl
from jax.experimental.pallas import tpu as pltpu

_HW = 32
_K, _S = 5, 2


def _avg_pool_matrix(n_in, n_out):
    m = np.zeros((n_out, n_in), dtype=np.float32)
    for o in range(n_out):
        s = (o * n_in) // n_out
        e = -(-((o + 1) * n_in) // n_out)
        m[o, s:e] = 1.0 / (e - s)
    return m


def _tail_matrix(hw):
    n1 = (hw - _K) // _S + 1
    n2 = (n1 - _K) // _S + 1
    a = _avg_pool_matrix(n2, hw // 8)
    f = np.zeros((hw, hw), dtype=np.float32)
    for p in range(hw):
        for b in range(n2):
            f[p, _S * _S * b] = a[p % (hw // 8), b]
    return f


def _kron_eye(m, k):
    return np.kron(np.eye(k, dtype=np.float32), m).astype(np.float32)


def _win13(x, axis):
    n = x.shape[axis]
    m = jnp.maximum(x, pltpu.roll(x, shift=n - 1, axis=axis))
    m = jnp.maximum(m, pltpu.roll(m, shift=n - 2, axis=axis))
    m = jnp.maximum(m, pltpu.roll(m, shift=n - 4, axis=axis))
    return jnp.maximum(m, pltpu.roll(m, shift=n - 5, axis=axis))


def _masks_kernel(skt_ref, pt_ref, bpt_ref, o_ref, *, B, hw):
    f32 = jnp.float32
    y = jnp.dot(skt_ref[0], pt_ref[...], preferred_element_type=f32)
    z = lax.dot_general(y, bpt_ref[...], (((0,), (0,)), ((), ())),
                        preferred_element_type=f32)
    lo = jnp.min(z)
    hi = jnp.max(z)
    r = (z - lo) / jnp.maximum(hi - lo, 1e-12)
    for b in range(B):
        o_ref[0, b * hw:(b + 1) * hw, :] = r[:, b * hw:(b + 1) * hw]


def _transfer_kernel(sf_ref, msk_ref, pt_ref, bpt_ref, f_ref, gt_ref, o_ref,
                     *, SPS, Cb, H, hw):
    f32 = jnp.float32
    m_t = jnp.concatenate([msk_ref[0]] * Cb, axis=1)
    cm_n = jnp.concatenate([jnp.swapaxes(msk_ref[1], 0, 1)] * Cb, axis=0)

    x = sf_ref[0]
    y = jnp.dot(x, pt_ref[...], preferred_element_type=f32)

    f_m = f_ref[...]
    gt_m = gt_ref[...]
    bpt = bpt_ref[...]
    for s in range(SPS):
        ys = y[s * Cb * H:(s + 1) * Cb * H]
        g = lax.dot_general(ys, bpt, (((0,), (0,)), ((), ())),
                            preferred_element_type=f32)
        e = g * m_t
        w = jnp.concatenate([e, g - e], axis=1)
        w = _win13(_win13(w, 1), 0)
        x1 = jnp.dot(w, gt_m, preferred_element_type=f32)
        v = lax.dot_general(x1, f_m, (((0,), (0,)), ((), ())),
                            preferred_element_type=f32)
        eo = v[:Cb * hw]
        po = v[Cb * hw:]
        o_ref[0, s] = po + (eo - po) * cm_n


@jax.jit
def _featmap_transfer(style_feat, style_skt, content_skt):
    B, C, H, W = style_feat.shape
    hw = _HW

    Cb = max(1, 128 // hw)
    while C % Cb:
        Cb //= 2
    nG = C // Cb

    max_sps = max(1, min(8, (2 * 1024 * 1024) // (Cb * H * W * 4)))
    if B == 1:
        max_sps = min(max_sps, max(1, nG // 2))
    SPS = 1
    for d in range(1, nG + 1):
        if nG % d == 0 and d <= max_sps:
            SPS = d
    nGsteps = nG // SPS

    p = _avg_pool_matrix(H, hw)
    f = _tail_matrix(hw)
    pt_f32 = jnp.asarray(p.T)
    bbt = jnp.asarray(_kron_eye(p.T, B))
    bpt_f32 = jnp.asarray(_kron_eye(p.T, Cb))
    ft_f32 = jnp.asarray(f.T)
    gt_f32 = jnp.asarray(_kron_eye(f.T, 2 * Cb))

    skts = jnp.stack([style_skt, content_skt], axis=0).reshape(2, B * H, W)
    masks = pl.pallas_call(
        functools.partial(_masks_kernel, B=B, hw=hw),
        out_shape=jax.ShapeDtypeStruct((2, B * hw, hw), jnp.float32),
        grid=(2,),
        in_specs=[pl.BlockSpec((1, B * H, W), lambda s: (s, 0, 0)),
                  pl.BlockSpec((H, hw), lambda s: (0, 0)),
                  pl.BlockSpec((B * H, B * hw), lambda s: (0, 0))],
        out_specs=pl.BlockSpec((1, B * hw, hw), lambda s: (s, 0, 0)),
        compiler_params=pltpu.CompilerParams(
            dimension_semantics=("parallel",)),
    )(skts, pt_f32, bbt)

    sf2 = style_feat.reshape(B, C * H, W)
    out_packed = pl.pallas_call(
        functools.partial(_transfer_kernel, SPS=SPS, Cb=Cb, H=H, hw=hw),
        out_shape=jax.ShapeDtypeStruct((B, nG, Cb * hw, hw), jnp.float32),
        grid=(B, nGsteps),
        in_specs=[
            pl.BlockSpec((1, SPS * Cb * H, W), lambda b, g: (b, g, 0)),
            pl.BlockSpec((2, hw, hw), lambda b, g: (0, b, 0)),
            pl.BlockSpec((H, hw), lambda b, g: (0, 0)),
            pl.BlockSpec((Cb * H, Cb * hw), lambda b, g: (0, 0)),
            pl.BlockSpec((hw, hw), lambda b, g: (0, 0)),
            pl.BlockSpec((2 * Cb * hw, 2 * Cb * hw), lambda b, g: (0, 0)),
        ],
        out_specs=pl.BlockSpec((1, SPS, Cb * hw, hw), lambda b, g: (b, g, 0, 0)),
        compiler_params=pltpu.CompilerParams(
            dimension_semantics=("parallel", "parallel"),
            vmem_limit_bytes=32 * 1024 * 1024),
    )(sf2, masks, pt_f32, bpt_f32, ft_f32, gt_f32)

    return out_packed.reshape(B, C, hw, hw)


def kernel(style_feat, style_skt, content_skt):
    return _featmap_transfer(style_feat, style_skt, content_skt)


# speedup vs baseline: 1.1821x; 1.1821x over previous
import functools

import numpy as np
import jax
import jax.numpy as jnp
from jax import lax
from jax.experimental import pallas as pl
from jax.experimental.pallas import tpu as pltpu

_HW = 32
_K, _S = 5, 2


def _avg_pool_matrix(n_in, n_out):
    m = np.zeros((n_out, n_in), dtype=np.float32)
    for o in range(n_out):
        s = (o * n_in) // n_out
        e = -(-((o + 1) * n_in) // n_out)
        m[o, s:e] = 1.0 / (e - s)
    return m


def _tail_matrix(hw):
    n1 = (hw - _K) // _S + 1
    n2 = (n1 - _K) // _S + 1
    a = _avg_pool_matrix(n2, hw // 8)
    f = np.zeros((hw, hw), dtype=np.float32)
    for p in range(hw):
        for b in range(n2):
            f[p, _S * _S * b] = a[p % (hw // 8), b]
    return f


def _kron_eye(m, k):
    return np.kron(np.eye(k, dtype=np.float32), m).astype(np.float32)


def _win13(x, axis):
    n = x.shape[axis]
    m = jnp.maximum(x, pltpu.roll(x, shift=n - 1, axis=axis))
    m = jnp.maximum(m, pltpu.roll(m, shift=n - 2, axis=axis))
    m = jnp.maximum(m, pltpu.roll(m, shift=n - 4, axis=axis))
    return jnp.maximum(m, pltpu.roll(m, shift=n - 5, axis=axis))


def _masks_kernel(skt_ref, pt_ref, bp_ref, o_ref):
    f32 = jnp.float32
    y = jnp.dot(skt_ref[0], pt_ref[...], preferred_element_type=f32)
    z = jnp.dot(bp_ref[...], y, preferred_element_type=f32)
    lo = jnp.min(z)
    hi = jnp.max(z)
    o_ref[0] = (z - lo) / jnp.maximum(hi - lo, 1e-12)


def _transfer_kernel(sf_ref, msk_ref, q_ref, ft2_ref, fblk_ref, o_ref,
                     *, NCH, Cb, hw):
    f32 = jnp.float32
    m_t = jnp.tile(msk_ref[0], (NCH, 1))
    cm_t = jnp.tile(msk_ref[1], (Cb, 1))

    g = jnp.dot(sf_ref[0], q_ref[...], preferred_element_type=f32)
    e = g * m_t
    w = jnp.concatenate([e, g - e], axis=1)
    w = _win13(_win13(w, 1), 0)
    w = jnp.dot(w, ft2_ref[...], preferred_element_type=f32)
    fb = fblk_ref[...]
    for k in range(NCH // Cb):
        r0 = k * Cb * hw
        vv = jnp.dot(fb, w[r0:r0 + Cb * hw],
                     preferred_element_type=f32)
        eo = vv[:, :hw]
        po = vv[:, hw:]
        o_ref[0, r0:r0 + Cb * hw, :] = po + (eo - po) * cm_t


@jax.jit
def _featmap_transfer(style_feat, style_skt, content_skt):
    B, C, H, W = style_feat.shape
    hw = _HW
    assert H == 2 * hw and W == 2 * hw, "kernel assumes exact 2x2 pooling"

    NCH = 32
    while C % NCH:
        NCH //= 2
    nsteps = C // NCH
    Cb = min(4, NCH)

    p = _avg_pool_matrix(H, hw)
    f = _tail_matrix(hw)
    pt_f32 = jnp.asarray(p.T)
    bp_f32 = jnp.asarray(_kron_eye(p, B))
    q_f32 = jnp.asarray(0.5 * np.vstack([p.T, p.T]))
    ft2_f32 = jnp.asarray(_kron_eye(f.T, 2))
    fblk_f32 = jnp.asarray(_kron_eye(f, Cb))

    skts = jnp.stack([style_skt, content_skt], axis=0).reshape(2, B * H, W)
    masks = pl.pallas_call(
        _masks_kernel,
        out_shape=jax.ShapeDtypeStruct((2, B * hw, hw), jnp.float32),
        grid=(2,),
        in_specs=[pl.BlockSpec((1, B * H, W), lambda s: (s, 0, 0)),
                  pl.BlockSpec((H, hw), lambda s: (0, 0)),
                  pl.BlockSpec((B * hw, B * H), lambda s: (0, 0))],
        out_specs=pl.BlockSpec((1, B * hw, hw), lambda s: (s, 0, 0)),
        compiler_params=pltpu.CompilerParams(
            dimension_semantics=("parallel",)),
    )(skts, pt_f32, bp_f32)

    sf2 = style_feat.reshape(B, C * H // 2, 2 * W)
    out_packed = pl.pallas_call(
        functools.partial(_transfer_kernel, NCH=NCH, Cb=Cb, hw=hw),
        out_shape=jax.ShapeDtypeStruct((B, C * hw, hw), jnp.float32),
        grid=(B, nsteps),
        in_specs=[
            pl.BlockSpec((1, NCH * hw, 2 * W), lambda b, g: (b, g, 0)),
            pl.BlockSpec((2, hw, hw), lambda b, g: (0, b, 0)),
            pl.BlockSpec((2 * W, hw), lambda b, g: (0, 0)),
            pl.BlockSpec((2 * hw, 2 * hw), lambda b, g: (0, 0)),
            pl.BlockSpec((Cb * hw, Cb * hw), lambda b, g: (0, 0)),
        ],
        out_specs=pl.BlockSpec((1, NCH * hw, hw), lambda b, g: (b, g, 0)),
        compiler_params=pltpu.CompilerParams(
            dimension_semantics=("parallel", "parallel"),
            vmem_limit_bytes=32 * 1024 * 1024),
    )(sf2, masks, q_f32, ft2_f32, fblk_f32)

    return out_packed.reshape(B, C, hw, hw)


def kernel(style_feat, style_skt, content_skt):
    return _featmap_transfer(style_feat, style_skt, content_skt)


# speedup vs baseline: 1.8195x; 1.5392x over previous
import functools

import numpy as np
import jax
import jax.numpy as jnp
from jax import lax
from jax.experimental import pallas as pl
from jax.experimental.pallas import tpu as pltpu

_HW = 32
_K, _S = 5, 2


def _avg_pool_matrix(n_in, n_out):
    m = np.zeros((n_out, n_in), dtype=np.float32)
    for o in range(n_out):
        s = (o * n_in) // n_out
        e = -(-((o + 1) * n_in) // n_out)
        m[o, s:e] = 1.0 / (e - s)
    return m


def _tail_matrix(hw):
    n1 = (hw - _K) // _S + 1
    n2 = (n1 - _K) // _S + 1
    a = _avg_pool_matrix(n2, hw // 8)
    f = np.zeros((hw, hw), dtype=np.float32)
    for p in range(hw):
        for b in range(n2):
            f[p, _S * _S * b] = a[p % (hw // 8), b]
    return f


def _kron_eye(m, k):
    return np.kron(np.eye(k, dtype=np.float32), m).astype(np.float32)


def _win13(x, axis):
    n = x.shape[axis]
    m = jnp.maximum(x, pltpu.roll(x, shift=n - 1, axis=axis))
    m = jnp.maximum(m, pltpu.roll(m, shift=n - 2, axis=axis))
    m = jnp.maximum(m, pltpu.roll(m, shift=n - 4, axis=axis))
    return jnp.maximum(m, pltpu.roll(m, shift=n - 5, axis=axis))


def _masks_kernel(skt_ref, pt_ref, bp_ref, o_ref):
    f32 = jnp.float32
    y = jnp.dot(skt_ref[0], pt_ref[...], preferred_element_type=f32)
    z = jnp.dot(bp_ref[...], y, preferred_element_type=f32)
    lo = jnp.min(z)
    hi = jnp.max(z)
    o_ref[0] = (z - lo) / jnp.maximum(hi - lo, 1e-12)


def _transfer_kernel(sf_ref, msk_ref, q_ref, ft_ref, fbt_ref, o_ref,
                     *, NCH, Cb, hw):
    f32 = jnp.float32
    m_t = jnp.tile(msk_ref[0], (NCH, 1))

    g = jnp.dot(sf_ref[0], q_ref[...], preferred_element_type=f32)
    e = g * m_t
    w = jnp.concatenate([e, g - e], axis=1)
    w = _win13(w, 0)
    t = jnp.swapaxes(w, 0, 1)
    t = _win13(t, 0)
    ft = ft_ref[...]
    u_e = lax.dot_general(t[:hw], ft, (((0,), (0,)), ((), ())),
                          preferred_element_type=f32)
    u_p = lax.dot_general(t[hw:], ft, (((0,), (0,)), ((), ())),
                          preferred_element_type=f32)
    u = jnp.concatenate([u_e, u_p], axis=1)
    fb = fbt_ref[...]
    cm_t = jnp.tile(msk_ref[1], (Cb, 1))
    for k in range(NCH // Cb):
        r0 = k * Cb * hw
        vv = jnp.dot(fb, u[r0:r0 + Cb * hw],
                     preferred_element_type=f32)
        eo = vv[:, :hw]
        po = vv[:, hw:]
        o_ref[0, r0:r0 + Cb * hw, :] = po + (eo - po) * cm_t


@jax.jit
def _featmap_transfer(style_feat, style_skt, content_skt):
    B, C, H, W = style_feat.shape
    hw = _HW
    assert H == 2 * hw and W == 2 * hw, "kernel assumes exact 2x2 pooling"

    NCH = 32
    while C % NCH:
        NCH //= 2
    nsteps = C // NCH
    Cb = min(4, NCH)

    p = _avg_pool_matrix(H, hw)
    f = _tail_matrix(hw)
    pt_f32 = jnp.asarray(p.T)
    bp_f32 = jnp.asarray(_kron_eye(p, B))
    q_f32 = jnp.asarray(0.5 * np.vstack([p.T, p.T]))
    ft_f32 = jnp.asarray(f.T)
    fbt_f32 = jnp.asarray(_kron_eye(f, Cb))

    skts = jnp.stack([style_skt, content_skt], axis=0).reshape(2, B * H, W)
    masks = pl.pallas_call(
        _masks_kernel,
        out_shape=jax.ShapeDtypeStruct((2, B * hw, hw), jnp.float32),
        grid=(2,),
        in_specs=[pl.BlockSpec((1, B * H, W), lambda s: (s, 0, 0)),
                  pl.BlockSpec((H, hw), lambda s: (0, 0)),
                  pl.BlockSpec((B * hw, B * H), lambda s: (0, 0))],
        out_specs=pl.BlockSpec((1, B * hw, hw), lambda s: (s, 0, 0)),
        compiler_params=pltpu.CompilerParams(
            dimension_semantics=("parallel",)),
    )(skts, pt_f32, bp_f32)

    sf2 = style_feat.reshape(B, C * H // 2, 2 * W)
    out_packed = pl.pallas_call(
        functools.partial(_transfer_kernel, NCH=NCH, Cb=Cb, hw=hw),
        out_shape=jax.ShapeDtypeStruct((B, C * hw, hw), jnp.float32),
        grid=(B, nsteps),
        in_specs=[
            pl.BlockSpec((1, NCH * hw, 2 * W), lambda b, g: (b, g, 0)),
            pl.BlockSpec((2, hw, hw), lambda b, g: (0, b, 0)),
            pl.BlockSpec((2 * W, hw), lambda b, g: (0, 0)),
            pl.BlockSpec((hw, hw), lambda b, g: (0, 0)),
            pl.BlockSpec((Cb * hw, Cb * hw), lambda b, g: (0, 0)),
        ],
        out_specs=pl.BlockSpec((1, NCH * hw, hw), lambda b, g: (b, g, 0)),
        compiler_params=pltpu.CompilerParams(
            dimension_semantics=("parallel", "parallel"),
            vmem_limit_bytes=32 * 1024 * 1024),
    )(sf2, masks, q_f32, ft_f32, fbt_f32)

    return out_packed.reshape(B, C, hw, hw)


def kernel(style_feat, style_skt, content_skt):
    return _featmap_transfer(style_feat, style_skt, content_skt)
